# baseline (speedup 1.0000x reference)
import functools

import jax
import jax.numpy as jnp
import numpy as np
from jax import lax
from jax.experimental import pallas as pl
from jax.experimental.pallas import tpu as pltpu
from jax.experimental.pallas import tpu_sc as plsc

N = 2048; E = 65536; C = 7; H = 128; SE = 20; D = H + SE; B = 4096; HEADS = 4
HD = 37; HDP = 40; DP = 160
NC = 2; NS = 16; L = 16
NW = NC * NS
EH = E // NC
ECH = 4096
NCH = EH // ECH
NG = ECH // L
EW = E // NW

_mesh = plsc.VectorSubcoreMesh(core_axis_name="c", subcore_axis_name="s")


def _zero_rows(ref, nrows):
    z = jnp.zeros((L,), jnp.float32)
    for r in range(nrows):
        @plsc.parallel_loop(0, N // L, unroll=4)
        def _zb(g, r=r):
            ref[r, pl.ds(g * L, L)] = z


def _zero_1d(ref):
    z = jnp.zeros((L,), jnp.float32)

    @plsc.parallel_loop(0, N // L, unroll=4)
    def _zb(g):
        ref[pl.ds(g * L, L)] = z


def _db_chunks(i, c, ei_hbm, sA, dA, sB, dB, semA, semB, process):
    def start(ch, sv, dv, sem):
        off = c * EH + ch * ECH
        pltpu.async_copy(ei_hbm.at[pl.ds(2 * i * E + off, ECH)], sv, sem)
        pltpu.async_copy(ei_hbm.at[pl.ds((2 * i + 1) * E + off, ECH)], dv, sem)

    def wait(ch, sv, dv, sem):
        off = c * EH + ch * ECH
        pltpu.make_async_copy(ei_hbm.at[pl.ds(2 * i * E + off, ECH)], sv, sem).wait()
        pltpu.make_async_copy(ei_hbm.at[pl.ds((2 * i + 1) * E + off, ECH)], dv, sem).wait()

    start(0, sA, dA, semA)

    def pbody(p, _):
        start(2 * p + 1, sB, dB, semB)
        wait(2 * p, sA, dA, semA)
        process(sA, dA)

        @pl.when(p + 1 < NCH // 2)
        def _():
            start(2 * p + 2, sA, dA, semA)
        wait(2 * p + 1, sB, dB, semB)
        process(sB, dB)
        return 0
    lax.fori_loop(0, NCH // 2, pbody, 0)


@functools.partial(
    pl.kernel,
    out_type=jax.ShapeDtypeStruct((C, NW, N), jnp.float32),
    mesh=_mesh,
    compiler_params=pltpu.CompilerParams(needs_layout_passes=False),
    scratch_types=[pltpu.VMEM((N,), jnp.float32), pltpu.VMEM((EW,), jnp.int32)],
)
def _k_deg(ei_hbm, out_hbm, acc_v, idx_v):
    c = lax.axis_index("c"); s = lax.axis_index("s")
    w = c * NS + s
    ones = jnp.ones((L,), jnp.float32)

    def rbody(i, _):
        _zero_1d(acc_v)
        pltpu.sync_copy(ei_hbm.at[pl.ds((2 * i + 1) * E + w * EW, EW)], idx_v)

        @plsc.parallel_loop(0, EW // L, unroll=4)
        def body(g):
            d16 = idx_v[pl.ds(g * L, L)]
            plsc.addupdate_scatter(acc_v, [d16], ones)
        pltpu.sync_copy(acc_v, out_hbm.at[i, w])
        return 0
    lax.fori_loop(0, C, rbody, 0)


def _make_k_gcn(lo, hi):
    @functools.partial(
        pl.kernel,
        out_type=jax.ShapeDtypeStruct((NC, hi - lo, NS, 10, N), jnp.float32),
        mesh=_mesh,
        compiler_params=pltpu.CompilerParams(needs_layout_passes=False),
        scratch_types=[pltpu.VMEM((10, N), jnp.float32), pltpu.VMEM((10, N), jnp.float32),
                       pltpu.VMEM((ECH,), jnp.int32), pltpu.VMEM((ECH,), jnp.int32),
                       pltpu.VMEM((ECH,), jnp.int32), pltpu.VMEM((ECH,), jnp.int32),
                       pltpu.SemaphoreType.DMA, pltpu.SemaphoreType.DMA],
        name=f"k_gcn_{lo}_{hi}",
    )
    def _k(tbl_hbm, ei_hbm, out_hbm, tbl_v, acc_v, sA, dA, sB, dB, semA, semB):
        c = lax.axis_index("c"); s = lax.axis_index("s")
        rows = [jnp.full((L,), r, jnp.int32) for r in range(10)]

        def rbody(i, _):
            pltpu.sync_copy(tbl_hbm.at[i, s], tbl_v)
            _zero_rows(acc_v, 10)

            def process(src_v, dst_v):
                @plsc.parallel_loop(0, NG, unroll=2)
                def gbody(g):
                    s16 = src_v[pl.ds(g * L, L)]
                    d16 = dst_v[pl.ds(g * L, L)]
                    for r in range(10):
                        v = plsc.load_gather(tbl_v, [rows[r], s16])
                        plsc.addupdate_scatter(acc_v, [rows[r], d16], v)
            _db_chunks(i, c, ei_hbm, sA, dA, sB, dB, semA, semB, process)
            pltpu.sync_copy(acc_v, out_hbm.at[c, i - lo, s])
            return 0
        lax.fori_loop(lo, hi, rbody, 0)
    return _k


_CSPLIT = 4
_k_gcn_a = _make_k_gcn(0, _CSPLIT)
_k_gcn_b = _make_k_gcn(_CSPLIT, C)


@functools.partial(
    pl.kernel,
    out_type=(jax.ShapeDtypeStruct((NC, C, NS, 8, N), jnp.float32),
              jax.ShapeDtypeStruct((NC * C * N,), jnp.float32)),
    mesh=_mesh,
    compiler_params=pltpu.CompilerParams(needs_layout_passes=False),
    scratch_types=[pltpu.VMEM((8, N), jnp.float32), pltpu.VMEM((8, N), jnp.float32),
                   pltpu.VMEM((N,), jnp.float32), pltpu.VMEM((N,), jnp.float32),
                   pltpu.VMEM((N,), jnp.float32), pltpu.VMEM((N,), jnp.float32),
                   pltpu.VMEM((N,), jnp.float32),
                   pltpu.VMEM((ECH,), jnp.int32), pltpu.VMEM((ECH,), jnp.int32),
                   pltpu.VMEM((ECH,), jnp.int32), pltpu.VMEM((ECH,), jnp.int32),
                   pltpu.SemaphoreType.DMA, pltpu.SemaphoreType.DMA],
)
def _k_gat(tbl_hbm, e1s_hbm, e1d_hbm, e2s_hbm, e2d_hbm, ei_hbm, out_hbm, den_hbm,
           tbl_v, acc_v, den_v, e1s_v, e1d_v, e2s_v, e2d_v, sA, dA, sB, dB, semA, semB):
    c = lax.axis_index("c"); s = lax.axis_index("s")
    rows = [jnp.full((L,), r, jnp.int32) for r in range(8)]

    def rbody(i, _):
        pltpu.sync_copy(tbl_hbm.at[i, s], tbl_v)
        pltpu.sync_copy(e1s_hbm.at[pl.ds(i * N, N)], e1s_v)
        pltpu.sync_copy(e1d_hbm.at[pl.ds(i * N, N)], e1d_v)
        pltpu.sync_copy(e2s_hbm.at[pl.ds(i * N, N)], e2s_v)
        pltpu.sync_copy(e2d_hbm.at[pl.ds(i * N, N)], e2d_v)
        _zero_rows(acc_v, 8)
        _zero_1d(den_v)

        def process(src_v, dst_v):
            @plsc.parallel_loop(0, NG, unroll=2)
            def gbody(g):
                s16 = src_v[pl.ds(g * L, L)]
                d16 = dst_v[pl.ds(g * L, L)]
                ex = jnp.maximum(
                    plsc.load_gather(e1s_v, [s16]) * plsc.load_gather(e1d_v, [d16]),
                    plsc.load_gather(e2s_v, [s16]) * plsc.load_gather(e2d_v, [d16]))
                for r in range(8):
                    v = plsc.load_gather(tbl_v, [rows[r], s16]) * ex
                    plsc.addupdate_scatter(acc_v, [rows[r], d16], v)

                @pl.when(s == 0)
                def _():
                    plsc.addupdate_scatter(den_v, [d16], ex)
        _db_chunks(i, c, ei_hbm, sA, dA, sB, dB, semA, semB, process)
        pltpu.sync_copy(acc_v, out_hbm.at[c, i, s])

        @pl.when(s == 0)
        def _():
            pltpu.sync_copy(den_v, den_hbm.at[pl.ds((c * C + i) * N, N)])
        return 0
    lax.fori_loop(0, C, rbody, 0)


@functools.partial(
    pl.kernel,
    out_type=jax.ShapeDtypeStruct((NC, C, NS, 16, N), jnp.float32),
    mesh=_mesh,
    compiler_params=pltpu.CompilerParams(needs_layout_passes=False),
    scratch_types=[pltpu.VMEM((16, N), jnp.float32), pltpu.VMEM((16, N), jnp.float32),
                   pltpu.VMEM((ECH,), jnp.int32), pltpu.VMEM((ECH,), jnp.int32),
                   pltpu.VMEM((ECH,), jnp.int32), pltpu.VMEM((ECH,), jnp.int32),
                   pltpu.SemaphoreType.DMA, pltpu.SemaphoreType.DMA],
)
def _k_sub(tbl_hbm, ei_hbm, out_hbm, tbl_v, acc_v, sA, dA, sB, dB, semA, semB):
    c = lax.axis_index("c"); s = lax.axis_index("s")
    rows = [jnp.full((L,), r, jnp.int32) for r in range(16)]

    def rbody(i, _):
        pltpu.sync_copy(tbl_hbm.at[i, s], tbl_v)
        _zero_rows(acc_v, 16)

        def process(src_v, dst_v):
            @plsc.parallel_loop(0, NG, unroll=2)
            def gbody(g):
                s16 = src_v[pl.ds(g * L, L)]
                d16 = dst_v[pl.ds(g * L, L)]
                for r in range(16):
                    v = plsc.load_gather(tbl_v, [rows[r], s16])
                    plsc.addupdate_scatter(acc_v, [rows[r], d16], v)
        _db_chunks(i, c, ei_hbm, sA, dA, sB, dB, semA, semB, process)
        pltpu.sync_copy(acc_v, out_hbm.at[c, i, s])
        return 0
    lax.fori_loop(0, C, rbody, 0)


RPT = B // NW
RCH = 32

@functools.partial(
    pl.kernel,
    out_type=jax.ShapeDtypeStruct((B, 8 * H), jnp.float32),
    mesh=_mesh,
    compiler_params=pltpu.CompilerParams(needs_layout_passes=False),
    scratch_types=[pltpu.VMEM((RPT,), jnp.int32), pltpu.VMEM((RPT,), jnp.int32),
                   pltpu.VMEM((RPT,), jnp.int32),
                   pltpu.VMEM((RCH, 8 * H), jnp.float32),
                   pltpu.VMEM((RCH, 8 * H), jnp.float32),
                   pltpu.SemaphoreType.DMA],
)
def _k_pair(xcat_hbm, ei0_hbm, ei1_hbm, tid_hbm, z_hbm,
            tid_v, n0_v, n1_v, ra_v, rb_v, sem):
    c = lax.axis_index("c"); s = lax.axis_index("s")
    w = c * NS + s
    base = w * RPT
    pltpu.sync_copy(tid_hbm.at[pl.ds(base, RPT)], tid_v)
    pltpu.async_copy(ei0_hbm.at[tid_v], n0_v, sem).wait()
    pltpu.async_copy(ei1_hbm.at[tid_v], n1_v, sem).wait()
    for j in range(RPT // RCH):
        pltpu.async_copy(xcat_hbm.at[n0_v.at[pl.ds(j * RCH, RCH)]], ra_v, sem).wait()
        pltpu.async_copy(xcat_hbm.at[n1_v.at[pl.ds(j * RCH, RCH)]], rb_v, sem).wait()
        for r in range(RCH):
            @plsc.parallel_loop(0, (8 * H) // L, unroll=4)
            def mbody(g, r=r):
                sl = (r, pl.ds(g * L, L))
                ra_v[sl] = ra_v[sl] * rb_v[sl]
        pltpu.sync_copy(ra_v, z_hbm.at[pl.ds(base + j * RCH, RCH)])


def _cls_body(z_ref, w0, w1, w2, w3, w4, w5, b0, b1, b2, b3, b4, b5, out_ref):
    z = z_ref[...]
    z = z @ w0[...] + b0[...]
    z = z @ w1[...] + b1[...]
    z = z @ w2[...] + b2[...]
    z = z @ w3[...] + b3[...]
    z = z @ w4[...] + b4[...]
    z = z @ w5[...] + b5[...]
    out_ref[...] = z


def _classifier(z, ws, bs):
    return pl.pallas_call(
        _cls_body,
        out_shape=jax.ShapeDtypeStruct((B, 7), jnp.float32),
    )(z, *ws, *[b.reshape(1, -1) for b in bs])


def _bn(xx, g, b, eps=1e-5):
    m = jnp.mean(xx, axis=0)
    v = jnp.var(xx, axis=0)
    return g * (xx - m) / jnp.sqrt(v + eps) + b


def _mha_ref(x, Wqkv, bqkv, Wo, bo, heads=HEADS):
    n, cc = x.shape
    hd = cc // heads
    q = (x @ Wqkv[0] + bqkv[0]).reshape(n, heads, hd).transpose(1, 0, 2)
    k = (x @ Wqkv[1] + bqkv[1]).reshape(n, heads, hd).transpose(1, 0, 2)
    v = (x @ Wqkv[2] + bqkv[2]).reshape(n, heads, hd).transpose(1, 0, 2)
    s = jnp.einsum('hqd,hkd->hqk', q, k) / np.sqrt(hd)
    a = jax.nn.softmax(s, axis=-1)
    o = jnp.einsum('hqk,hkd->hqd', a, v).transpose(1, 0, 2).reshape(n, cc)
    return o @ Wo + bo


def kernel(x, se, seven_edge_index, edge_index, train_edge_id, gps_gcn_w, gps_gcn_b, attn_qkv_w, attn_qkv_b, attn_out_w, attn_out_b, mlp_w1, mlp_b1, mlp_w2, mlp_b2, gps_bn_g, gps_bn_b, lin_w, lin_b, gat_w, gat_b, gat_att_src, gat_att_dst, sub_w, sub_b, gcn2_w, gcn2_b, blk_bn_g, blk_bn_b, cls_w0, cls_b0, cls_w1, cls_b1, cls_w2, cls_b2, cls_w3, cls_b3, cls_w4, cls_b4, cls_w5, cls_b5):
    ei3 = seven_edge_index
    ei = seven_edge_index.reshape(-1)

    degpart = _k_deg(ei)
    cnt_all = degpart.sum(axis=1)
    deg = cnt_all + 1.0
    dinv_all = lax.rsqrt(deg)
    dinv2_all = 1.0 / deg

    g1T_list, hg_list, t_list = [], [], []
    for i in range(C):
        t = jnp.concatenate([x, se[i]], axis=1)
        hg = t @ gps_gcn_w[i]
        g1T_list.append((hg * dinv_all[i][:, None]).T)
        hg_list.append(hg)
        t_list.append(t)
    g1T = jnp.pad(jnp.stack(g1T_list), ((0, 0), (0, DP - D), (0, 0)))

    g1Tr = g1T.reshape(C, NS, 10, N)
    part1a = _k_gcn_a(g1Tr[:_CSPLIT], ei)
    part1b = _k_gcn_b(g1Tr, ei)
    part1 = jnp.concatenate([part1a, part1b], axis=1).reshape(NC, C, DP, N)
    gsum = (part1[0] + part1[1])[:, :D]

    h_list, t2_list, hgat_list = [], [], []
    e1s_l, e1d_l, e2s_l, e2d_l = [], [], [], []
    for i in range(C):
        t = t_list[i]; hg = hg_list[i]
        dinv = dinv_all[i]; dinv2 = dinv2_all[i]
        gcn1 = gsum[i].T * dinv[:, None] + hg * dinv2[:, None] + gps_gcn_b[i]
        h1 = _bn(gcn1 + t, gps_bn_g[i, 0], gps_bn_b[i, 0])
        h2 = _mha_ref(t, attn_qkv_w[i], attn_qkv_b[i], attn_out_w[i], attn_out_b[i])
        h2 = _bn(h2 + t, gps_bn_g[i, 1], gps_bn_b[i, 1])
        h = h1 + h2
        h = h + (jax.nn.relu(h @ mlp_w1[i] + mlp_b1[i]) @ mlp_w2[i] + mlp_b2[i])
        h = _bn(h, gps_bn_g[i, 2], gps_bn_b[i, 2])
        t2 = h @ lin_w[i] + lin_b[i]
        hgat = t2 @ gat_w[i]
        hs = hgat @ gat_att_src[i]
        hd = hgat @ gat_att_dst[i]
        a_sh = jnp.maximum(hs.max(), 0.0)
        b_sh = jnp.maximum(hd.max(), 0.0)
        e1s_l.append(jnp.exp(hs - a_sh)); e1d_l.append(jnp.exp(hd - b_sh))
        e2s_l.append(jnp.exp(0.2 * hs - a_sh)); e2d_l.append(jnp.exp(0.2 * hd - b_sh))
        hgat_list.append(hgat)
    e1s = jnp.stack(e1s_l); e1d = jnp.stack(e1d_l)
    e2s = jnp.stack(e2s_l); e2d = jnp.stack(e2d_l)
    hgatT = jnp.stack([hh.T for hh in hgat_list])

    msgpart, denpart = _k_gat(hgatT.reshape(C, NS, 8, N), e1s.reshape(-1),
                              e1d.reshape(-1), e2s.reshape(-1), e2d.reshape(-1), ei)
    msgpart = msgpart.reshape(NC, C, H, N)
    denpart = denpart.reshape(NC, C, N)

    t3_list, h2g_list, catT_list = [], [], []
    for i in range(C):
        hgat = hgat_list[i]
        ex_self = jnp.maximum(e1s[i] * e1d[i], e2s[i] * e2d[i])
        den = denpart[0, i] + denpart[1, i] + ex_self
        msg = (msgpart[0, i] + msgpart[1, i]).T + ex_self[:, None] * hgat
        t3 = msg / den[:, None] + gat_b[i]
        h2g = t3 @ gcn2_w[i]
        t3_list.append(t3); h2g_list.append(h2g)
        catT_list.append(jnp.concatenate([t3.T, (h2g * dinv_all[i][:, None]).T], axis=0))
    catT = jnp.stack(catT_list)

    part3 = _k_sub(catT.reshape(C, NS, 16, N), ei)
    part3 = part3.reshape(NC, C, 2 * H, N)
    s3 = part3[0] + part3[1]

    outs = [x]
    for i in range(C):
        subsum = s3[i, :H].T
        g2sum = s3[i, H:].T
        mean = subsum / jnp.maximum(cnt_all[i], 1.0)[:, None]
        t_sub = jax.nn.relu(mean @ sub_w[i] + sub_b[i])
        tt = (g2sum * dinv_all[i][:, None] + h2g_list[i] * dinv2_all[i][:, None]
              + gcn2_b[i])
        outs.append(_bn(tt + t_sub, blk_bn_g[i], blk_bn_b[i]))

    xcat = jnp.concatenate(outs, axis=1)
    z = _k_pair(xcat, edge_index[0], edge_index[1], train_edge_id)
    return _classifier(z, (cls_w0, cls_w1, cls_w2, cls_w3, cls_w4, cls_w5),
                       (cls_b0, cls_b1, cls_b2, cls_b3, cls_b4, cls_b5))


# speedup vs baseline: 1.0322x; 1.0322x over previous
import functools

import jax
import jax.numpy as jnp
import numpy as np
from jax import lax
from jax.experimental import pallas as pl
from jax.experimental.pallas import tpu as pltpu
from jax.experimental.pallas import tpu_sc as plsc

N = 2048; E = 65536; C = 7; H = 128; SE = 20; D = H + SE; B = 4096; HEADS = 4
HD = 37; HDP = 40; DP = 160
NC = 2; NS = 16; L = 16
NW = NC * NS
EH = E // NC
ECH = 4096
NCH = EH // ECH
NG = ECH // L
EW = E // NW

_mesh = plsc.VectorSubcoreMesh(core_axis_name="c", subcore_axis_name="s")


def _zero_rows(ref, nrows):
    z = jnp.zeros((L,), jnp.float32)
    for r in range(nrows):
        @plsc.parallel_loop(0, N // L, unroll=4)
        def _zb(g, r=r):
            ref[r, pl.ds(g * L, L)] = z


def _zero_1d(ref):
    z = jnp.zeros((L,), jnp.float32)

    @plsc.parallel_loop(0, N // L, unroll=4)
    def _zb(g):
        ref[pl.ds(g * L, L)] = z


def _db_chunks(i, c, ei_hbm, sA, dA, sB, dB, semA, semB, process):
    def start(ch, sv, dv, sem):
        off = c * EH + ch * ECH
        pltpu.async_copy(ei_hbm.at[pl.ds(2 * i * E + off, ECH)], sv, sem)
        pltpu.async_copy(ei_hbm.at[pl.ds((2 * i + 1) * E + off, ECH)], dv, sem)

    def wait(ch, sv, dv, sem):
        off = c * EH + ch * ECH
        pltpu.make_async_copy(ei_hbm.at[pl.ds(2 * i * E + off, ECH)], sv, sem).wait()
        pltpu.make_async_copy(ei_hbm.at[pl.ds((2 * i + 1) * E + off, ECH)], dv, sem).wait()

    start(0, sA, dA, semA)

    def pbody(p, _):
        start(2 * p + 1, sB, dB, semB)
        wait(2 * p, sA, dA, semA)
        process(sA, dA)

        @pl.when(p + 1 < NCH // 2)
        def _():
            start(2 * p + 2, sA, dA, semA)
        wait(2 * p + 1, sB, dB, semB)
        process(sB, dB)
        return 0
    lax.fori_loop(0, NCH // 2, pbody, 0)


@functools.partial(
    pl.kernel,
    out_type=jax.ShapeDtypeStruct((C, NW, N), jnp.float32),
    mesh=_mesh,
    compiler_params=pltpu.CompilerParams(needs_layout_passes=False),
    scratch_types=[pltpu.VMEM((N,), jnp.float32), pltpu.VMEM((EW,), jnp.int32)],
)
def _k_deg(ei_hbm, out_hbm, acc_v, idx_v):
    c = lax.axis_index("c"); s = lax.axis_index("s")
    w = c * NS + s
    ones = jnp.ones((L,), jnp.float32)

    def rbody(i, _):
        _zero_1d(acc_v)
        pltpu.sync_copy(ei_hbm.at[pl.ds((2 * i + 1) * E + w * EW, EW)], idx_v)

        @plsc.parallel_loop(0, EW // L, unroll=4)
        def body(g):
            d16 = idx_v[pl.ds(g * L, L)]
            plsc.addupdate_scatter(acc_v, [d16], ones)
        pltpu.sync_copy(acc_v, out_hbm.at[i, w])
        return 0
    lax.fori_loop(0, C, rbody, 0)


@functools.partial(
    pl.kernel,
    out_type=jax.ShapeDtypeStruct((NC, C, NS, 10, N), jnp.float32),
    mesh=_mesh,
    compiler_params=pltpu.CompilerParams(needs_layout_passes=False),
    scratch_types=[pltpu.VMEM((10, N), jnp.float32), pltpu.VMEM((10, N), jnp.float32),
                   pltpu.VMEM((ECH,), jnp.int32), pltpu.VMEM((ECH,), jnp.int32),
                   pltpu.VMEM((ECH,), jnp.int32), pltpu.VMEM((ECH,), jnp.int32),
                   pltpu.SemaphoreType.DMA, pltpu.SemaphoreType.DMA],
)
def _k_gcn(tbl_hbm, ei_hbm, out_hbm, tbl_v, acc_v, sA, dA, sB, dB, semA, semB):
    c = lax.axis_index("c"); s = lax.axis_index("s")
    rows = [jnp.full((L,), r, jnp.int32) for r in range(10)]

    def rbody(i, _):
        pltpu.sync_copy(tbl_hbm.at[i, s], tbl_v)
        _zero_rows(acc_v, 10)

        def process(src_v, dst_v):
            @plsc.parallel_loop(0, NG, unroll=2)
            def gbody(g):
                s16 = src_v[pl.ds(g * L, L)]
                d16 = dst_v[pl.ds(g * L, L)]
                for r in range(10):
                    v = plsc.load_gather(tbl_v, [rows[r], s16])
                    plsc.addupdate_scatter(acc_v, [rows[r], d16], v)
        _db_chunks(i, c, ei_hbm, sA, dA, sB, dB, semA, semB, process)
        pltpu.sync_copy(acc_v, out_hbm.at[c, i, s])
        return 0
    lax.fori_loop(0, C, rbody, 0)


@functools.partial(
    pl.kernel,
    out_type=(jax.ShapeDtypeStruct((NC, C, NS, 8, N), jnp.float32),
              jax.ShapeDtypeStruct((NC * C * N,), jnp.float32)),
    mesh=_mesh,
    compiler_params=pltpu.CompilerParams(needs_layout_passes=False),
    scratch_types=[pltpu.VMEM((8, N), jnp.float32), pltpu.VMEM((8, N), jnp.float32),
                   pltpu.VMEM((N,), jnp.float32), pltpu.VMEM((N,), jnp.float32),
                   pltpu.VMEM((N,), jnp.float32), pltpu.VMEM((N,), jnp.float32),
                   pltpu.VMEM((N,), jnp.float32),
                   pltpu.VMEM((ECH,), jnp.int32), pltpu.VMEM((ECH,), jnp.int32),
                   pltpu.VMEM((ECH,), jnp.int32), pltpu.VMEM((ECH,), jnp.int32),
                   pltpu.SemaphoreType.DMA, pltpu.SemaphoreType.DMA],
)
def _k_gat(tbl_hbm, e1s_hbm, e1d_hbm, e2s_hbm, e2d_hbm, ei_hbm, out_hbm, den_hbm,
           tbl_v, acc_v, den_v, e1s_v, e1d_v, e2s_v, e2d_v, sA, dA, sB, dB, semA, semB):
    c = lax.axis_index("c"); s = lax.axis_index("s")
    rows = [jnp.full((L,), r, jnp.int32) for r in range(8)]

    def rbody(i, _):
        pltpu.sync_copy(tbl_hbm.at[i, s], tbl_v)
        pltpu.sync_copy(e1s_hbm.at[pl.ds(i * N, N)], e1s_v)
        pltpu.sync_copy(e1d_hbm.at[pl.ds(i * N, N)], e1d_v)
        pltpu.sync_copy(e2s_hbm.at[pl.ds(i * N, N)], e2s_v)
        pltpu.sync_copy(e2d_hbm.at[pl.ds(i * N, N)], e2d_v)
        _zero_rows(acc_v, 8)
        _zero_1d(den_v)

        def process(src_v, dst_v):
            @plsc.parallel_loop(0, NG, unroll=2)
            def gbody(g):
                s16 = src_v[pl.ds(g * L, L)]
                d16 = dst_v[pl.ds(g * L, L)]
                ex = jnp.maximum(
                    plsc.load_gather(e1s_v, [s16]) * plsc.load_gather(e1d_v, [d16]),
                    plsc.load_gather(e2s_v, [s16]) * plsc.load_gather(e2d_v, [d16]))
                for r in range(8):
                    v = plsc.load_gather(tbl_v, [rows[r], s16]) * ex
                    plsc.addupdate_scatter(acc_v, [rows[r], d16], v)

                @pl.when(s == 0)
                def _():
                    plsc.addupdate_scatter(den_v, [d16], ex)
        _db_chunks(i, c, ei_hbm, sA, dA, sB, dB, semA, semB, process)
        pltpu.sync_copy(acc_v, out_hbm.at[c, i, s])

        @pl.when(s == 0)
        def _():
            pltpu.sync_copy(den_v, den_hbm.at[pl.ds((c * C + i) * N, N)])
        return 0
    lax.fori_loop(0, C, rbody, 0)


@functools.partial(
    pl.kernel,
    out_type=jax.ShapeDtypeStruct((NC, C, NS, 16, N), jnp.float32),
    mesh=_mesh,
    compiler_params=pltpu.CompilerParams(needs_layout_passes=False),
    scratch_types=[pltpu.VMEM((16, N), jnp.float32), pltpu.VMEM((16, N), jnp.float32),
                   pltpu.VMEM((ECH,), jnp.int32), pltpu.VMEM((ECH,), jnp.int32),
                   pltpu.VMEM((ECH,), jnp.int32), pltpu.VMEM((ECH,), jnp.int32),
                   pltpu.SemaphoreType.DMA, pltpu.SemaphoreType.DMA],
)
def _k_sub(tbl_hbm, ei_hbm, out_hbm, tbl_v, acc_v, sA, dA, sB, dB, semA, semB):
    c = lax.axis_index("c"); s = lax.axis_index("s")
    rows = [jnp.full((L,), r, jnp.int32) for r in range(16)]

    def rbody(i, _):
        pltpu.sync_copy(tbl_hbm.at[i, s], tbl_v)
        _zero_rows(acc_v, 16)

        def process(src_v, dst_v):
            @plsc.parallel_loop(0, NG, unroll=2)
            def gbody(g):
                s16 = src_v[pl.ds(g * L, L)]
                d16 = dst_v[pl.ds(g * L, L)]
                for r in range(16):
                    v = plsc.load_gather(tbl_v, [rows[r], s16])
                    plsc.addupdate_scatter(acc_v, [rows[r], d16], v)
        _db_chunks(i, c, ei_hbm, sA, dA, sB, dB, semA, semB, process)
        pltpu.sync_copy(acc_v, out_hbm.at[c, i, s])
        return 0
    lax.fori_loop(0, C, rbody, 0)


RPT = B // NW
RCH = 32

@functools.partial(
    pl.kernel,
    out_type=jax.ShapeDtypeStruct((B, 8 * H), jnp.float32),
    mesh=_mesh,
    compiler_params=pltpu.CompilerParams(needs_layout_passes=False),
    scratch_types=[pltpu.VMEM((RPT,), jnp.int32), pltpu.VMEM((RPT,), jnp.int32),
                   pltpu.VMEM((RPT,), jnp.int32),
                   pltpu.VMEM((RCH, 8 * H), jnp.float32),
                   pltpu.VMEM((RCH, 8 * H), jnp.float32),
                   pltpu.SemaphoreType.DMA],
)
def _k_pair(xcat_hbm, ei0_hbm, ei1_hbm, tid_hbm, z_hbm,
            tid_v, n0_v, n1_v, ra_v, rb_v, sem):
    c = lax.axis_index("c"); s = lax.axis_index("s")
    w = c * NS + s
    base = w * RPT
    pltpu.sync_copy(tid_hbm.at[pl.ds(base, RPT)], tid_v)
    pltpu.async_copy(ei0_hbm.at[tid_v], n0_v, sem).wait()
    pltpu.async_copy(ei1_hbm.at[tid_v], n1_v, sem).wait()
    for j in range(RPT // RCH):
        pltpu.async_copy(xcat_hbm.at[n0_v.at[pl.ds(j * RCH, RCH)]], ra_v, sem).wait()
        pltpu.async_copy(xcat_hbm.at[n1_v.at[pl.ds(j * RCH, RCH)]], rb_v, sem).wait()
        for r in range(RCH):
            @plsc.parallel_loop(0, (8 * H) // L, unroll=4)
            def mbody(g, r=r):
                sl = (r, pl.ds(g * L, L))
                ra_v[sl] = ra_v[sl] * rb_v[sl]
        pltpu.sync_copy(ra_v, z_hbm.at[pl.ds(base + j * RCH, RCH)])


def _cls_body(z_ref, w0, w1, w2, w3, w4, w5, b0, b1, b2, b3, b4, b5, out_ref):
    z = z_ref[...]
    z = z @ w0[...] + b0[...]
    z = z @ w1[...] + b1[...]
    z = z @ w2[...] + b2[...]
    z = z @ w3[...] + b3[...]
    z = z @ w4[...] + b4[...]
    z = z @ w5[...] + b5[...]
    out_ref[...] = z


def _classifier(z, ws, bs):
    return pl.pallas_call(
        _cls_body,
        out_shape=jax.ShapeDtypeStruct((B, 7), jnp.float32),
    )(z, *ws, *[b.reshape(1, -1) for b in bs])


def _bn(xx, g, b, eps=1e-5):
    m = jnp.mean(xx, axis=0)
    v = jnp.var(xx, axis=0)
    return g * (xx - m) / jnp.sqrt(v + eps) + b


def _mha_ref(x, Wqkv, bqkv, Wo, bo, heads=HEADS):
    n, cc = x.shape
    hd = cc // heads
    q = (x @ Wqkv[0] + bqkv[0]).reshape(n, heads, hd).transpose(1, 0, 2)
    k = (x @ Wqkv[1] + bqkv[1]).reshape(n, heads, hd).transpose(1, 0, 2)
    v = (x @ Wqkv[2] + bqkv[2]).reshape(n, heads, hd).transpose(1, 0, 2)
    s = jnp.einsum('hqd,hkd->hqk', q, k) / np.sqrt(hd)
    a = jax.nn.softmax(s, axis=-1)
    o = jnp.einsum('hqk,hkd->hqd', a, v).transpose(1, 0, 2).reshape(n, cc)
    return o @ Wo + bo


def kernel(x, se, seven_edge_index, edge_index, train_edge_id, gps_gcn_w, gps_gcn_b, attn_qkv_w, attn_qkv_b, attn_out_w, attn_out_b, mlp_w1, mlp_b1, mlp_w2, mlp_b2, gps_bn_g, gps_bn_b, lin_w, lin_b, gat_w, gat_b, gat_att_src, gat_att_dst, sub_w, sub_b, gcn2_w, gcn2_b, blk_bn_g, blk_bn_b, cls_w0, cls_b0, cls_w1, cls_b1, cls_w2, cls_b2, cls_w3, cls_b3, cls_w4, cls_b4, cls_w5, cls_b5):
    ei3 = seven_edge_index
    ei = seven_edge_index.reshape(-1)

    degpart = _k_deg(ei)
    cnt_all = degpart.sum(axis=1)
    deg = cnt_all + 1.0
    dinv_all = lax.rsqrt(deg)
    dinv2_all = 1.0 / deg

    g1T_list, hg_list, t_list = [], [], []
    for i in range(C):
        t = jnp.concatenate([x, se[i]], axis=1)
        hg = t @ gps_gcn_w[i]
        g1T_list.append((hg * dinv_all[i][:, None]).T)
        hg_list.append(hg)
        t_list.append(t)
    g1T = jnp.pad(jnp.stack(g1T_list), ((0, 0), (0, DP - D), (0, 0)))

    part1 = _k_gcn(g1T.reshape(C, NS, 10, N), ei)
    part1 = part1.reshape(NC, C, DP, N)
    gsum = (part1[0] + part1[1])[:, :D]

    h_list, t2_list, hgat_list = [], [], []
    e1s_l, e1d_l, e2s_l, e2d_l = [], [], [], []
    for i in range(C):
        t = t_list[i]; hg = hg_list[i]
        dinv = dinv_all[i]; dinv2 = dinv2_all[i]
        gcn1 = gsum[i].T * dinv[:, None] + hg * dinv2[:, None] + gps_gcn_b[i]
        h1 = _bn(gcn1 + t, gps_bn_g[i, 0], gps_bn_b[i, 0])
        h2 = _mha_ref(t, attn_qkv_w[i], attn_qkv_b[i], attn_out_w[i], attn_out_b[i])
        h2 = _bn(h2 + t, gps_bn_g[i, 1], gps_bn_b[i, 1])
        h = h1 + h2
        h = h + (jax.nn.relu(h @ mlp_w1[i] + mlp_b1[i]) @ mlp_w2[i] + mlp_b2[i])
        h = _bn(h, gps_bn_g[i, 2], gps_bn_b[i, 2])
        t2 = h @ lin_w[i] + lin_b[i]
        hgat = t2 @ gat_w[i]
        hs = hgat @ gat_att_src[i]
        hd = hgat @ gat_att_dst[i]
        a_sh = jnp.maximum(hs.max(), 0.0)
        b_sh = jnp.maximum(hd.max(), 0.0)
        e1s_l.append(jnp.exp(hs - a_sh)); e1d_l.append(jnp.exp(hd - b_sh))
        e2s_l.append(jnp.exp(0.2 * hs - a_sh)); e2d_l.append(jnp.exp(0.2 * hd - b_sh))
        hgat_list.append(hgat)
    e1s = jnp.stack(e1s_l); e1d = jnp.stack(e1d_l)
    e2s = jnp.stack(e2s_l); e2d = jnp.stack(e2d_l)
    hgatT = jnp.stack([hh.T for hh in hgat_list])

    msgpart, denpart = _k_gat(hgatT.reshape(C, NS, 8, N), e1s.reshape(-1),
                              e1d.reshape(-1), e2s.reshape(-1), e2d.reshape(-1), ei)
    msgpart = msgpart.reshape(NC, C, H, N)
    denpart = denpart.reshape(NC, C, N)

    t3_list, h2g_list, catT_list = [], [], []
    for i in range(C):
        hgat = hgat_list[i]
        ex_self = jnp.maximum(e1s[i] * e1d[i], e2s[i] * e2d[i])
        den = denpart[0, i] + denpart[1, i] + ex_self
        msg = (msgpart[0, i] + msgpart[1, i]).T + ex_self[:, None] * hgat
        t3 = msg / den[:, None] + gat_b[i]
        h2g = t3 @ gcn2_w[i]
        t3_list.append(t3); h2g_list.append(h2g)
        catT_list.append(jnp.concatenate([t3.T, (h2g * dinv_all[i][:, None]).T], axis=0))
    catT = jnp.stack(catT_list)

    part3 = _k_sub(catT.reshape(C, NS, 16, N), ei)
    part3 = part3.reshape(NC, C, 2 * H, N)
    s3 = part3[0] + part3[1]

    outs = [x]
    for i in range(C):
        subsum = s3[i, :H].T
        g2sum = s3[i, H:].T
        mean = subsum / jnp.maximum(cnt_all[i], 1.0)[:, None]
        t_sub = jax.nn.relu(mean @ sub_w[i] + sub_b[i])
        tt = (g2sum * dinv_all[i][:, None] + h2g_list[i] * dinv2_all[i][:, None]
              + gcn2_b[i])
        outs.append(_bn(tt + t_sub, blk_bn_g[i], blk_bn_b[i]))

    xcat = jnp.concatenate(outs, axis=1)
    z = _k_pair(xcat, edge_index[0], edge_index[1], train_edge_id)
    return _classifier(z, (cls_w0, cls_w1, cls_w2, cls_w3, cls_w4, cls_w5),
                       (cls_b0, cls_b1, cls_b2, cls_b3, cls_b4, cls_b5))
